# Initial kernel scaffold; baseline (speedup 1.0000x reference)
import jax
import jax.numpy as jnp
from jax.experimental import pallas as pl


def kernel(indices, logits):
    raise NotImplementedError("write your pallas kernel here")


# speedup vs baseline: 1.1824x; 1.1824x over previous
import functools

import jax
import jax.numpy as jnp
from jax import lax
from jax.experimental import pallas as pl
from jax.experimental.pallas import tpu as pltpu
from jax.experimental.pallas import tpu_sc as plsc


def kernel(indices, logits):
    B, = indices.shape
    V, D = logits.shape
    info = plsc.get_sparse_core_info()
    NC, NS, L = info.num_cores, info.num_subcores, info.num_lanes
    NW = NC * NS
    b_per_w = B // NW
    mesh = plsc.VectorSubcoreMesh(core_axis_name="c", subcore_axis_name="s")

    @functools.partial(
        pl.kernel,
        mesh=mesh,
        out_type=jax.ShapeDtypeStruct((B, D), jnp.float32),
        scratch_types=[
            pltpu.VMEM((b_per_w,), jnp.int32),
            pltpu.VMEM((b_per_w, D), jnp.float32),
            pltpu.SemaphoreType.DMA,
        ],
    )
    def _run(idx_hbm, table_hbm, out_hbm, idx_v, rows_v, sem):
        wid = lax.axis_index("s") * NC + lax.axis_index("c")
        base = wid * b_per_w
        pltpu.sync_copy(idx_hbm.at[pl.ds(base, b_per_w)], idx_v)
        pltpu.async_copy(table_hbm.at[idx_v], rows_v, sem).wait()

        def body(r, carry):
            for c in range(D // L):
                x = rows_v[r, pl.ds(c * L, L)]
                rows_v[r, pl.ds(c * L, L)] = 1.0 / (1.0 + jnp.exp(-x))
            return carry

        lax.fori_loop(0, b_per_w, body, 0)
        pltpu.sync_copy(rows_v, out_hbm.at[pl.ds(base, b_per_w)])

    return _run(indices, logits)


# speedup vs baseline: 1.3592x; 1.1496x over previous
import functools

import jax
import jax.numpy as jnp
from jax import lax
from jax.experimental import pallas as pl
from jax.experimental.pallas import tpu as pltpu
from jax.experimental.pallas import tpu_sc as plsc

_CHUNK = 128


def kernel(indices, logits):
    B, = indices.shape
    V, D = logits.shape
    info = plsc.get_sparse_core_info()
    NC, NS, L = info.num_cores, info.num_subcores, info.num_lanes
    NW = NC * NS
    b_per_w = B // NW
    n_chunks = b_per_w // _CHUNK
    mesh = plsc.VectorSubcoreMesh(core_axis_name="c", subcore_axis_name="s")

    @functools.partial(
        pl.kernel,
        mesh=mesh,
        out_type=jax.ShapeDtypeStruct((B, D), jnp.float32),
        scratch_types=[
            pltpu.VMEM((b_per_w,), jnp.int32),
            pltpu.VMEM((_CHUNK, D), jnp.float32),
            pltpu.VMEM((_CHUNK, D), jnp.float32),
            pltpu.SemaphoreType.DMA,
            pltpu.SemaphoreType.DMA,
            pltpu.SemaphoreType.DMA,
            pltpu.SemaphoreType.DMA,
        ],
    )
    def _run(idx_hbm, table_hbm, out_hbm, idx_v, buf0, buf1, g0, g1, w0, w1):
        wid = lax.axis_index("s") * NC + lax.axis_index("c")
        base = wid * b_per_w
        bufs, gsems, wsems = (buf0, buf1), (g0, g1), (w0, w1)
        pltpu.sync_copy(idx_hbm.at[pl.ds(base, b_per_w)], idx_v)

        def start_gather(g):
            s = g % 2
            return pltpu.async_copy(
                table_hbm.at[idx_v.at[pl.ds(g * _CHUNK, _CHUNK)]],
                bufs[s], gsems[s])

        gcopies = [None] * n_chunks
        wcopies = [None] * n_chunks
        gcopies[0] = start_gather(0)
        for g in range(n_chunks):
            s = g % 2
            if g + 1 < n_chunks:
                if g >= 1:
                    wcopies[g - 1].wait()
                gcopies[g + 1] = start_gather(g + 1)
            gcopies[g].wait()
            buf = bufs[s]

            @plsc.parallel_loop(0, _CHUNK, unroll=4)
            def _sigmoid_rows(r):
                for c in range(D // L):
                    x = buf[r, pl.ds(c * L, L)]
                    buf[r, pl.ds(c * L, L)] = 1.0 / (1.0 + jnp.exp(-x))

            wcopies[g] = pltpu.async_copy(
                buf, out_hbm.at[pl.ds(base + g * _CHUNK, _CHUNK)], wsems[s])
        for g in range(max(0, n_chunks - 2), n_chunks):
            wcopies[g].wait()

    return _run(indices, logits)
